# baseline (speedup 1.0000x reference)
import functools

import jax
import jax.numpy as jnp
from jax import lax
from jax.experimental import pallas as pl
from jax.experimental.pallas import tpu as pltpu
from jax.experimental.pallas import tpu_sc as plsc

N = 10000
E = 320000
NC = 2
NS = 16
NW = NC * NS
EW = E // NW
CH = 125
NCH = EW // CH
NB = 4
NG = NCH // NB
RU = 80
NRU = N // RU


def _seg_body(D, do_gather, spmem_gather, nb, g_hbm, src_hbm, dst_hbm, out_hbm,
              src_v, dst_v, rows_v, acc, gtab, gsem, ssem):
    c = lax.axis_index("c")
    s = lax.axis_index("s")
    w = c * NS + s

    nunit = -(-NRU // NS)
    for k in range(nunit):
        j = s + k * NS

        @pl.when(j < NRU)
        def _():
            pltpu.async_copy(g_hbm.at[pl.ds(j * RU, RU)],
                             acc.at[pl.ds(j * RU, RU)], gsem.at[0, 0])
            if spmem_gather:
                pltpu.async_copy(g_hbm.at[pl.ds(j * RU, RU)],
                                 gtab.at[pl.ds(j * RU, RU)], gsem.at[0, 3])
    pltpu.async_copy(dst_hbm.at[w], dst_v, gsem.at[0, 1])
    if do_gather:
        pltpu.async_copy(src_hbm.at[w], src_v, gsem.at[0, 2])
        pltpu.make_async_copy(src_hbm.at[w], src_v, gsem.at[0, 2]).wait()
    else:
        pltpu.async_copy(g_hbm.at[pl.ds(0, CH)], rows_v.at[0, 0],
                         gsem.at[0, 2])
        pltpu.make_async_copy(g_hbm.at[pl.ds(0, CH)], rows_v.at[0, 0],
                              gsem.at[0, 2]).wait()
    pltpu.make_async_copy(dst_hbm.at[w], dst_v, gsem.at[0, 1]).wait()
    for k in range(nunit):
        j = s + k * NS

        @pl.when(j < NRU)
        def _():
            pltpu.make_async_copy(g_hbm.at[pl.ds(j * RU, RU)],
                                  acc.at[pl.ds(j * RU, RU)],
                                  gsem.at[0, 0]).wait()
            if spmem_gather:
                pltpu.make_async_copy(g_hbm.at[pl.ds(j * RU, RU)],
                                      gtab.at[pl.ds(j * RU, RU)],
                                      gsem.at[0, 3]).wait()
    plsc.subcore_barrier()

    if do_gather:
        gsrc = gtab if spmem_gather else g_hbm
        for i in range(nb):
            pltpu.async_copy(gsrc.at[src_v.at[i]], rows_v.at[0, i],
                             gsem.at[0, i])

        def body(k, carry):
            h = lax.rem(k, 2)
            nh = lax.rem(k + 1, 2)
            for i in range(nb):
                j = k * nb + i
                pltpu.make_async_copy(gsrc.at[src_v.at[j]], rows_v.at[h, i],
                                      gsem.at[h, i]).wait()
                pltpu.async_copy(rows_v.at[h, i], acc.at[dst_v.at[j]],
                                 ssem.at[i], add=True)

            @pl.when(k + 1 < NCH // nb)
            def _():
                for i in range(nb):
                    j = (k + 1) * nb + i
                    pltpu.async_copy(gsrc.at[src_v.at[j]], rows_v.at[nh, i],
                                     gsem.at[nh, i])

            for i in range(nb):
                j = k * nb + i
                pltpu.make_async_copy(rows_v.at[h, i], acc.at[dst_v.at[j]],
                                      ssem.at[i]).wait()
            return carry

        lax.fori_loop(0, NCH // nb, body, 0)
    else:
        def body(k, carry):
            for i in range(nb):
                j = k * nb + i
                pltpu.async_copy(rows_v.at[0, 0], acc.at[dst_v.at[j]],
                                 ssem.at[i], add=True)
            for i in range(nb):
                j = k * nb + i
                pltpu.make_async_copy(rows_v.at[0, 0], acc.at[dst_v.at[j]],
                                      ssem.at[i]).wait()
            return carry

        lax.fori_loop(0, NCH // nb, body, 0)
    plsc.subcore_barrier()

    for k in range(nunit):
        j = s + k * NS

        @pl.when(j < NRU)
        def _():
            pltpu.async_copy(acc.at[pl.ds(j * RU, RU)],
                             out_hbm.at[c, pl.ds(j * RU, RU)], gsem.at[0, 0])
    for k in range(nunit):
        j = s + k * NS

        @pl.when(j < NRU)
        def _():
            pltpu.make_async_copy(acc.at[pl.ds(j * RU, RU)],
                                  out_hbm.at[c, pl.ds(j * RU, RU)],
                                  gsem.at[0, 0]).wait()


def _make_seg(D, do_gather=True, spmem_gather=False, nb=NB):
    mesh = plsc.VectorSubcoreMesh(core_axis_name="c", subcore_axis_name="s")
    rows_shape = (2, nb, CH, D) if do_gather else (1, 1, CH, D)
    gtab_shape = (N, D) if spmem_gather else (8, D)
    return pl.kernel(
        functools.partial(_seg_body, D, do_gather, spmem_gather, nb),
        out_type=jax.ShapeDtypeStruct((NC, N, D), jnp.float32),
        mesh=mesh,
        scratch_types=[
            pltpu.VMEM((NCH, CH), jnp.int32),
            pltpu.VMEM((NCH, CH), jnp.int32),
            pltpu.VMEM(rows_shape, jnp.float32),
            pltpu.VMEM_SHARED((N, D), jnp.float32),
            pltpu.VMEM_SHARED(gtab_shape, jnp.float32),
            pltpu.SemaphoreType.DMA((2, nb)),
            pltpu.SemaphoreType.DMA((nb,)),
        ],
        compiler_params=pltpu.CompilerParams(use_tc_tiling_on_sc=False),
    )


_seg16 = _make_seg(16, spmem_gather=True)
_seg64 = _make_seg(64)

RT = N // NS
RTP = 640


def _rsqrt16(x):
    i = plsc.bitcast(x, jnp.int32)
    i = jnp.full((16,), 0x5F3759DF, jnp.int32) - jnp.right_shift(i, 1)
    y = plsc.bitcast(i, jnp.float32)
    for _ in range(3):
        y = y * (1.5 - 0.5 * x * y * y)
    return y


def _mega1_body(h1_hbm, src_hbm, dst_hbm, s1p_hbm, g1_hbm, dinv_hbm,
                src_v, dst_v, rows_v, ones_v, degv, h1v, g1v, dinv_v,
                acc_deg, acc_s1, gtab, gsem, ssem):
    c = lax.axis_index("c")
    s = lax.axis_index("s")
    w = c * NS + s
    iota16 = lax.iota(jnp.int32, 16)
    zeros16 = jnp.zeros((16,), jnp.int32)

    for r in range(CH):
        ones_v[r, :] = jnp.zeros((16,), jnp.float32)
    nzu = -(-NRU // NS)
    for k in range(nzu):
        j = s + k * NS

        @pl.when(j < NRU)
        def _():
            pltpu.async_copy(ones_v.at[pl.ds(0, RU)],
                             acc_deg.at[pl.ds(j * RU, RU)], gsem.at[0, 0])
    pltpu.async_copy(dst_hbm.at[s], dst_v.at[pl.ds(0, NCH)], gsem.at[0, 1])
    pltpu.async_copy(dst_hbm.at[NS + s], dst_v.at[pl.ds(NCH, NCH)],
                     gsem.at[0, 2])
    pltpu.async_copy(src_hbm.at[w], src_v, gsem.at[0, 3])
    for k in range(nzu):
        j = s + k * NS

        @pl.when(j < NRU)
        def _():
            pltpu.make_async_copy(ones_v.at[pl.ds(0, RU)],
                                  acc_deg.at[pl.ds(j * RU, RU)],
                                  gsem.at[0, 0]).wait()
    for r in range(CH):
        ones_v[r, :] = jnp.ones((16,), jnp.float32)
    pltpu.make_async_copy(dst_hbm.at[s], dst_v.at[pl.ds(0, NCH)],
                          gsem.at[0, 1]).wait()
    pltpu.make_async_copy(dst_hbm.at[NS + s], dst_v.at[pl.ds(NCH, NCH)],
                          gsem.at[0, 2]).wait()
    plsc.subcore_barrier()

    def deg_body(k, carry):
        for i in range(NB):
            j = k * NB + i
            pltpu.async_copy(ones_v, acc_deg.at[dst_v.at[j]], ssem.at[i],
                             add=True)
        for i in range(NB):
            j = k * NB + i
            pltpu.make_async_copy(ones_v, acc_deg.at[dst_v.at[j]],
                                  ssem.at[i]).wait()
        return carry

    lax.fori_loop(0, (2 * NCH) // NB, deg_body, 0)
    plsc.subcore_barrier()

    pltpu.async_copy(acc_deg.at[pl.ds(s * RT, RT)], degv.at[pl.ds(0, RT)],
                     gsem.at[0, 0])
    pltpu.async_copy(h1_hbm.at[pl.ds(s * RT, RT)], h1v.at[pl.ds(0, RT)],
                     gsem.at[0, 1])
    pltpu.make_async_copy(acc_deg.at[pl.ds(s * RT, RT)],
                          degv.at[pl.ds(0, RT)], gsem.at[0, 0]).wait()
    pltpu.make_async_copy(h1_hbm.at[pl.ds(s * RT, RT)],
                          h1v.at[pl.ds(0, RT)], gsem.at[0, 1]).wait()

    def dinv_body(v, carry):
        d = plsc.load_gather(degv, [v * 16 + iota16, zeros16])
        dinv_v[pl.ds(v * 16, 16)] = _rsqrt16(d + 1.0)
        return carry

    lax.fori_loop(0, RTP // 16, dinv_body, 0)

    def g1_body(r, carry):
        rsp = jnp.full((16,), r, jnp.int32)
        dv = plsc.load_gather(dinv_v, [rsp])
        hrow = plsc.load_gather(h1v, [rsp, iota16])
        plsc.store_scatter(g1v, [rsp, iota16], dv * hrow)
        return carry

    lax.fori_loop(0, RT, g1_body, 0)

    pltpu.async_copy(g1v.at[pl.ds(0, RT)], gtab.at[pl.ds(s * RT, RT)],
                     gsem.at[0, 0])
    pltpu.async_copy(g1v.at[pl.ds(0, RT)], acc_s1.at[pl.ds(s * RT, RT)],
                     gsem.at[0, 1])
    pltpu.async_copy(dinv_v, dinv_hbm.at[c, s], gsem.at[0, 3])

    @pl.when(c == 0)
    def _():
        pltpu.async_copy(g1v.at[pl.ds(0, RT)], g1_hbm.at[pl.ds(s * RT, RT)],
                         gsem.at[0, 2])
        pltpu.make_async_copy(g1v.at[pl.ds(0, RT)],
                              g1_hbm.at[pl.ds(s * RT, RT)],
                              gsem.at[0, 2]).wait()

    pltpu.make_async_copy(g1v.at[pl.ds(0, RT)], gtab.at[pl.ds(s * RT, RT)],
                          gsem.at[0, 0]).wait()
    pltpu.make_async_copy(g1v.at[pl.ds(0, RT)], acc_s1.at[pl.ds(s * RT, RT)],
                          gsem.at[0, 1]).wait()
    pltpu.make_async_copy(dinv_v, dinv_hbm.at[c, s], gsem.at[0, 3]).wait()
    plsc.subcore_barrier()

    woff = c * NCH
    for i in range(NB):
        pltpu.async_copy(gtab.at[src_v.at[i]], rows_v.at[0, i], gsem.at[0, i])

    def edge_body(k, carry):
        h = lax.rem(k, 2)
        nh = lax.rem(k + 1, 2)
        for i in range(NB):
            j = k * NB + i
            pltpu.make_async_copy(gtab.at[src_v.at[j]], rows_v.at[h, i],
                                  gsem.at[h, i]).wait()
            pltpu.async_copy(rows_v.at[h, i], acc_s1.at[dst_v.at[woff + j]],
                             ssem.at[i], add=True)

        @pl.when(k + 1 < NG)
        def _():
            for i in range(NB):
                j = (k + 1) * NB + i
                pltpu.async_copy(gtab.at[src_v.at[j]], rows_v.at[nh, i],
                                 gsem.at[nh, i])

        for i in range(NB):
            j = k * NB + i
            pltpu.make_async_copy(rows_v.at[h, i], acc_s1.at[dst_v.at[woff + j]],
                                  ssem.at[i]).wait()
        return carry

    lax.fori_loop(0, NG, edge_body, 0)
    plsc.subcore_barrier()

    for k in range(nzu):
        j = s + k * NS

        @pl.when(j < NRU)
        def _():
            pltpu.async_copy(acc_s1.at[pl.ds(j * RU, RU)],
                             s1p_hbm.at[c, pl.ds(j * RU, RU)], gsem.at[0, 0])
    for k in range(nzu):
        j = s + k * NS

        @pl.when(j < NRU)
        def _():
            pltpu.make_async_copy(acc_s1.at[pl.ds(j * RU, RU)],
                                  s1p_hbm.at[c, pl.ds(j * RU, RU)],
                                  gsem.at[0, 0]).wait()


_mega1 = pl.kernel(
    _mega1_body,
    out_type=(jax.ShapeDtypeStruct((NC, N, 16), jnp.float32),
              jax.ShapeDtypeStruct((N, 16), jnp.float32),
              jax.ShapeDtypeStruct((NC, NS, RTP), jnp.float32)),
    mesh=plsc.VectorSubcoreMesh(core_axis_name="c", subcore_axis_name="s"),
    scratch_types=[
        pltpu.VMEM((NCH, CH), jnp.int32),
        pltpu.VMEM((2 * NCH, CH), jnp.int32),
        pltpu.VMEM((2, NB, CH, 16), jnp.float32),
        pltpu.VMEM((CH, 16), jnp.float32),
        pltpu.VMEM((RTP, 16), jnp.float32),
        pltpu.VMEM((RTP, 16), jnp.float32),
        pltpu.VMEM((RTP, 16), jnp.float32),
        pltpu.VMEM((RTP,), jnp.float32),
        pltpu.VMEM_SHARED((N, 16), jnp.float32),
        pltpu.VMEM_SHARED((N, 16), jnp.float32),
        pltpu.VMEM_SHARED((N, 16), jnp.float32),
        pltpu.SemaphoreType.DMA((2, NB)),
        pltpu.SemaphoreType.DMA((NB,)),
    ],
    compiler_params=pltpu.CompilerParams(use_tc_tiling_on_sc=False,
                                         needs_layout_passes=False),
)


def _tc_call(body, out_shapes):
    return pl.pallas_call(body, out_shape=out_shapes)


def _tc_a1_body(x, w1, h1_o):
    h1_o[...] = jnp.dot(x[...], w1[...], preferred_element_type=jnp.float32)


def _tc_a2_body(degp, h1, dinv_o, g1_o):
    deg = degp[0, :, 0:1] + degp[1, :, 0:1] - 1.0
    dinv = lax.rsqrt(deg)
    dinv_o[...] = dinv
    g1_o[...] = dinv * h1[...]


def _tc_b_body(s1p, g1, dinv, b1, g2_o):
    t = dinv[...] * (s1p[0] + s1p[1] - g1[...])
    z1 = jnp.maximum(t + b1[...], 0.0)
    g2_o[...] = dinv[...] * z1


def _tc_c_body(s2p, g2, dinv, w2, b2, g3_o):
    t = dinv[...] * (s2p[0] + s2p[1] - g2[...])
    z2 = jnp.maximum(jnp.dot(t, w2[...], preferred_element_type=jnp.float32)
                     + b2[...], 0.0)
    g3_o[...] = dinv[...] * z2


def _tc_d_body(s3p, g3, dinv, w3, b3, wfc, bfc, out_o):
    t = dinv[...] * (s3p[0] + s3p[1] - g3[...])
    z3 = jnp.maximum(jnp.dot(t, w3[...], preferred_element_type=jnp.float32)
                     + b3[...], 0.0)
    out_o[...] = jnp.dot(z3, wfc[...],
                         preferred_element_type=jnp.float32) + bfc[...]


def kernel(x, edge_index, W1, b1, W2, b2, W3, b3, Wfc, bfc):
    src3 = edge_index[0].reshape(NW, NCH, CH)
    dst3 = edge_index[1].reshape(NW, NCH, CH)

    h1 = _tc_call(
        _tc_a1_body,
        jax.ShapeDtypeStruct((N, 16), jnp.float32))(x, W1)
    s1p, g1, dinvp = _mega1(h1, src3, dst3)
    dinv = dinvp[0][:, :RT].reshape(N, 1)
    g2 = _tc_call(
        _tc_b_body,
        jax.ShapeDtypeStruct((N, 16), jnp.float32))(
            s1p, g1, dinv, b1.reshape(1, 16))

    s2p = _seg16(g2, src3, dst3)
    g3 = _tc_call(
        _tc_c_body,
        jax.ShapeDtypeStruct((N, 64), jnp.float32))(
            s2p, g2, dinv, W2, b2.reshape(1, 64))

    s3p = _seg64(g3, src3, dst3)
    out = _tc_call(
        _tc_d_body,
        jax.ShapeDtypeStruct((N, 1), jnp.float32))(
            s3p, g3, dinv, W3, b3.reshape(1, 128), Wfc, bfc.reshape(1, 1))
    return out


# speedup vs baseline: 1.0211x; 1.0211x over previous
import functools

import jax
import jax.numpy as jnp
from jax import lax
from jax.experimental import pallas as pl
from jax.experimental.pallas import tpu as pltpu
from jax.experimental.pallas import tpu_sc as plsc

N = 10000
E = 320000
NC = 2
NS = 16
NW = NC * NS
EW = E // NW
CH = 125
NCH = EW // CH
NB = 4
NG = NCH // NB
RU = 80
NRU = N // RU


def _seg_body(D, do_gather, spmem_gather, nb, g_hbm, src_hbm, dst_hbm, out_hbm,
              src_v, dst_v, rows_v, acc, gtab, gsem, ssem):
    c = lax.axis_index("c")
    s = lax.axis_index("s")
    w = c * NS + s

    nunit = -(-NRU // NS)
    for k in range(nunit):
        j = s + k * NS

        @pl.when(j < NRU)
        def _():
            pltpu.async_copy(g_hbm.at[pl.ds(j * RU, RU)],
                             acc.at[pl.ds(j * RU, RU)], gsem.at[0, 0])
            if spmem_gather:
                pltpu.async_copy(g_hbm.at[pl.ds(j * RU, RU)],
                                 gtab.at[pl.ds(j * RU, RU)], gsem.at[0, 3])
    pltpu.async_copy(dst_hbm.at[w], dst_v, gsem.at[0, 1])
    if do_gather:
        pltpu.async_copy(src_hbm.at[w], src_v, gsem.at[0, 2])
        pltpu.make_async_copy(src_hbm.at[w], src_v, gsem.at[0, 2]).wait()
    else:
        pltpu.async_copy(g_hbm.at[pl.ds(0, CH)], rows_v.at[0, 0],
                         gsem.at[0, 2])
        pltpu.make_async_copy(g_hbm.at[pl.ds(0, CH)], rows_v.at[0, 0],
                              gsem.at[0, 2]).wait()
    pltpu.make_async_copy(dst_hbm.at[w], dst_v, gsem.at[0, 1]).wait()
    for k in range(nunit):
        j = s + k * NS

        @pl.when(j < NRU)
        def _():
            pltpu.make_async_copy(g_hbm.at[pl.ds(j * RU, RU)],
                                  acc.at[pl.ds(j * RU, RU)],
                                  gsem.at[0, 0]).wait()
            if spmem_gather:
                pltpu.make_async_copy(g_hbm.at[pl.ds(j * RU, RU)],
                                      gtab.at[pl.ds(j * RU, RU)],
                                      gsem.at[0, 3]).wait()
    plsc.subcore_barrier()

    if do_gather:
        gsrc = gtab if spmem_gather else g_hbm
        for i in range(nb):
            pltpu.async_copy(gsrc.at[src_v.at[i]], rows_v.at[0, i],
                             gsem.at[0, i])

        def body(k, carry):
            h = lax.rem(k, 2)
            nh = lax.rem(k + 1, 2)
            for i in range(nb):
                j = k * nb + i
                pltpu.make_async_copy(gsrc.at[src_v.at[j]], rows_v.at[h, i],
                                      gsem.at[h, i]).wait()
                pltpu.async_copy(rows_v.at[h, i], acc.at[dst_v.at[j]],
                                 ssem.at[i], add=True)

            @pl.when(k + 1 < NCH // nb)
            def _():
                for i in range(nb):
                    j = (k + 1) * nb + i
                    pltpu.async_copy(gsrc.at[src_v.at[j]], rows_v.at[nh, i],
                                     gsem.at[nh, i])

            for i in range(nb):
                j = k * nb + i
                pltpu.make_async_copy(rows_v.at[h, i], acc.at[dst_v.at[j]],
                                      ssem.at[i]).wait()
            return carry

        lax.fori_loop(0, NCH // nb, body, 0)
    else:
        def body(k, carry):
            for i in range(nb):
                j = k * nb + i
                pltpu.async_copy(rows_v.at[0, 0], acc.at[dst_v.at[j]],
                                 ssem.at[i], add=True)
            for i in range(nb):
                j = k * nb + i
                pltpu.make_async_copy(rows_v.at[0, 0], acc.at[dst_v.at[j]],
                                      ssem.at[i]).wait()
            return carry

        lax.fori_loop(0, NCH // nb, body, 0)
    plsc.subcore_barrier()

    for k in range(nunit):
        j = s + k * NS

        @pl.when(j < NRU)
        def _():
            pltpu.async_copy(acc.at[pl.ds(j * RU, RU)],
                             out_hbm.at[c, pl.ds(j * RU, RU)], gsem.at[0, 0])
    for k in range(nunit):
        j = s + k * NS

        @pl.when(j < NRU)
        def _():
            pltpu.make_async_copy(acc.at[pl.ds(j * RU, RU)],
                                  out_hbm.at[c, pl.ds(j * RU, RU)],
                                  gsem.at[0, 0]).wait()


def _make_seg(D, do_gather=True, spmem_gather=False, nb=NB):
    mesh = plsc.VectorSubcoreMesh(core_axis_name="c", subcore_axis_name="s")
    rows_shape = (2, nb, CH, D) if do_gather else (1, 1, CH, D)
    gtab_shape = (N, D) if spmem_gather else (8, D)
    return pl.kernel(
        functools.partial(_seg_body, D, do_gather, spmem_gather, nb),
        out_type=jax.ShapeDtypeStruct((NC, N, D), jnp.float32),
        mesh=mesh,
        scratch_types=[
            pltpu.VMEM((NCH, CH), jnp.int32),
            pltpu.VMEM((NCH, CH), jnp.int32),
            pltpu.VMEM(rows_shape, jnp.float32),
            pltpu.VMEM_SHARED((N, D), jnp.float32),
            pltpu.VMEM_SHARED(gtab_shape, jnp.float32),
            pltpu.SemaphoreType.DMA((2, nb)),
            pltpu.SemaphoreType.DMA((nb,)),
        ],
        compiler_params=pltpu.CompilerParams(use_tc_tiling_on_sc=False),
    )


_seg16 = _make_seg(16, spmem_gather=True)
_seg64 = _make_seg(64)

_seg8_const = _make_seg(8, do_gather=False)


def _tc_call(body, out_shapes):
    return pl.pallas_call(body, out_shape=out_shapes)


def _tc_a1_body(x, w1, h1_o):
    h1_o[...] = jnp.dot(x[...], w1[...], preferred_element_type=jnp.float32)


def _tc_a2_body(degp, h1, dinv_o, g1_o):
    deg = degp[0, :, 0:1] + degp[1, :, 0:1] - 1.0
    dinv = lax.rsqrt(deg)
    dinv_o[...] = dinv
    g1_o[...] = dinv * h1[...]


def _tc_b_body(s1p, g1, dinv, b1, g2_o):
    t = dinv[...] * (s1p[0] + s1p[1] - g1[...])
    z1 = jnp.maximum(t + b1[...], 0.0)
    g2_o[...] = dinv[...] * z1


def _tc_c_body(s2p, g2, dinv, w2, b2, g3_o):
    t = dinv[...] * (s2p[0] + s2p[1] - g2[...])
    z2 = jnp.maximum(jnp.dot(t, w2[...], preferred_element_type=jnp.float32)
                     + b2[...], 0.0)
    g3_o[...] = dinv[...] * z2


def _tc_d_body(s3p, g3, dinv, w3, b3, wfc, bfc, out_o):
    t = dinv[...] * (s3p[0] + s3p[1] - g3[...])
    z3 = jnp.maximum(jnp.dot(t, w3[...], preferred_element_type=jnp.float32)
                     + b3[...], 0.0)
    out_o[...] = jnp.dot(z3, wfc[...],
                         preferred_element_type=jnp.float32) + bfc[...]


def kernel(x, edge_index, W1, b1, W2, b2, W3, b3, Wfc, bfc):
    src3 = edge_index[0].reshape(NW, NCH, CH)
    dst3 = edge_index[1].reshape(NW, NCH, CH)

    ones8 = jnp.ones((N, 8), dtype=jnp.float32)
    degp = _seg8_const(ones8, src3, dst3)

    def _tc_a_merged(degp_r, x_r, w1_r, dinv_o, g1_o):
        deg = degp_r[0, :, 0:1] + degp_r[1, :, 0:1] - 1.0
        dinv = lax.rsqrt(deg)
        dinv_o[...] = dinv
        g1_o[...] = dinv * jnp.dot(x_r[...], w1_r[...],
                                   preferred_element_type=jnp.float32)

    dinv, g1 = _tc_call(
        _tc_a_merged,
        (jax.ShapeDtypeStruct((N, 1), jnp.float32),
         jax.ShapeDtypeStruct((N, 16), jnp.float32)))(degp, x, W1)

    s1p = _seg16(g1, src3, dst3)
    g2 = _tc_call(
        _tc_b_body,
        jax.ShapeDtypeStruct((N, 16), jnp.float32))(
            s1p, g1, dinv, b1.reshape(1, 16))

    s2p = _seg16(g2, src3, dst3)
    g3 = _tc_call(
        _tc_c_body,
        jax.ShapeDtypeStruct((N, 64), jnp.float32))(
            s2p, g2, dinv, W2, b2.reshape(1, 64))

    s3p = _seg64(g3, src3, dst3)
    out = _tc_call(
        _tc_d_body,
        jax.ShapeDtypeStruct((N, 1), jnp.float32))(
            s3p, g3, dinv, W3, b3.reshape(1, 128), Wfc, bfc.reshape(1, 1))
    return out


# speedup vs baseline: 1.0211x; 1.0000x over previous
import functools

import jax
import jax.numpy as jnp
from jax import lax
from jax.experimental import pallas as pl
from jax.experimental.pallas import tpu as pltpu
from jax.experimental.pallas import tpu_sc as plsc

N = 10000
E = 320000
NC = 2
NS = 16
NW = NC * NS
EW = E // NW
CH = 125
NCH = EW // CH
NB = 4
NG = NCH // NB
RU = 80
NRU = N // RU


def _seg_body(D, do_gather, spmem_gather, nb, g_hbm, src_hbm, dst_hbm, out_hbm,
              src_v, dst_v, rows_v, acc, gtab, gsem, ssem):
    c = lax.axis_index("c")
    s = lax.axis_index("s")
    w = c * NS + s

    nunit = -(-NRU // NS)
    for k in range(nunit):
        j = s + k * NS

        @pl.when(j < NRU)
        def _():
            pltpu.async_copy(g_hbm.at[pl.ds(j * RU, RU)],
                             acc.at[pl.ds(j * RU, RU)], gsem.at[0, 0])
            if spmem_gather:
                pltpu.async_copy(g_hbm.at[pl.ds(j * RU, RU)],
                                 gtab.at[pl.ds(j * RU, RU)], gsem.at[0, 3])
    pltpu.async_copy(dst_hbm.at[w], dst_v, gsem.at[0, 1])
    if do_gather:
        pltpu.async_copy(src_hbm.at[w], src_v, gsem.at[0, 2])
        pltpu.make_async_copy(src_hbm.at[w], src_v, gsem.at[0, 2]).wait()
    else:
        pltpu.async_copy(g_hbm.at[pl.ds(0, CH)], rows_v.at[0, 0],
                         gsem.at[0, 2])
        pltpu.make_async_copy(g_hbm.at[pl.ds(0, CH)], rows_v.at[0, 0],
                              gsem.at[0, 2]).wait()
    pltpu.make_async_copy(dst_hbm.at[w], dst_v, gsem.at[0, 1]).wait()
    for k in range(nunit):
        j = s + k * NS

        @pl.when(j < NRU)
        def _():
            pltpu.make_async_copy(g_hbm.at[pl.ds(j * RU, RU)],
                                  acc.at[pl.ds(j * RU, RU)],
                                  gsem.at[0, 0]).wait()
            if spmem_gather:
                pltpu.make_async_copy(g_hbm.at[pl.ds(j * RU, RU)],
                                      gtab.at[pl.ds(j * RU, RU)],
                                      gsem.at[0, 3]).wait()
    plsc.subcore_barrier()

    if do_gather:
        gsrc = gtab if spmem_gather else g_hbm
        for i in range(nb):
            pltpu.async_copy(gsrc.at[src_v.at[i]], rows_v.at[0, i],
                             gsem.at[0, i])

        def body(k, carry):
            h = lax.rem(k, 2)
            nh = lax.rem(k + 1, 2)
            for i in range(nb):
                j = k * nb + i
                pltpu.make_async_copy(gsrc.at[src_v.at[j]], rows_v.at[h, i],
                                      gsem.at[h, i]).wait()
                pltpu.async_copy(rows_v.at[h, i], acc.at[dst_v.at[j]],
                                 ssem.at[i], add=True)

            @pl.when(k + 1 < NCH // nb)
            def _():
                for i in range(nb):
                    j = (k + 1) * nb + i
                    pltpu.async_copy(gsrc.at[src_v.at[j]], rows_v.at[nh, i],
                                     gsem.at[nh, i])

            for i in range(nb):
                j = k * nb + i
                pltpu.make_async_copy(rows_v.at[h, i], acc.at[dst_v.at[j]],
                                      ssem.at[i]).wait()
            return carry

        lax.fori_loop(0, NCH // nb, body, 0)
    else:
        def body(k, carry):
            for i in range(nb):
                j = k * nb + i
                pltpu.async_copy(rows_v.at[0, 0], acc.at[dst_v.at[j]],
                                 ssem.at[i], add=True)
            for i in range(nb):
                j = k * nb + i
                pltpu.make_async_copy(rows_v.at[0, 0], acc.at[dst_v.at[j]],
                                      ssem.at[i]).wait()
            return carry

        lax.fori_loop(0, NCH // nb, body, 0)
    plsc.subcore_barrier()

    for k in range(nunit):
        j = s + k * NS

        @pl.when(j < NRU)
        def _():
            pltpu.async_copy(acc.at[pl.ds(j * RU, RU)],
                             out_hbm.at[c, pl.ds(j * RU, RU)], gsem.at[0, 0])
    for k in range(nunit):
        j = s + k * NS

        @pl.when(j < NRU)
        def _():
            pltpu.make_async_copy(acc.at[pl.ds(j * RU, RU)],
                                  out_hbm.at[c, pl.ds(j * RU, RU)],
                                  gsem.at[0, 0]).wait()


def _make_seg(D, do_gather=True, spmem_gather=False, nb=NB):
    mesh = plsc.VectorSubcoreMesh(core_axis_name="c", subcore_axis_name="s")
    rows_shape = (2, nb, CH, D) if do_gather else (1, 1, CH, D)
    gtab_shape = (N, D) if spmem_gather else (8, D)
    return pl.kernel(
        functools.partial(_seg_body, D, do_gather, spmem_gather, nb),
        out_type=jax.ShapeDtypeStruct((NC, N, D), jnp.float32),
        mesh=mesh,
        scratch_types=[
            pltpu.VMEM((NCH, CH), jnp.int32),
            pltpu.VMEM((NCH, CH), jnp.int32),
            pltpu.VMEM(rows_shape, jnp.float32),
            pltpu.VMEM_SHARED((N, D), jnp.float32),
            pltpu.VMEM_SHARED(gtab_shape, jnp.float32),
            pltpu.SemaphoreType.DMA((2, nb)),
            pltpu.SemaphoreType.DMA((nb,)),
        ],
        compiler_params=pltpu.CompilerParams(use_tc_tiling_on_sc=False),
    )


_seg16 = _make_seg(16, spmem_gather=True)
_seg64 = _make_seg(64)

_seg8_const = _make_seg(8, do_gather=False)


def _tc_call(body, out_shapes):
    return pl.pallas_call(body, out_shape=out_shapes)


def _tc_a_body(degp, x, w1, dinv_o, g1_o):
    deg = degp[0, :, 0:1] + degp[1, :, 0:1] - 1.0
    dinv = lax.rsqrt(deg)
    dinv_o[...] = dinv
    g1_o[...] = dinv * jnp.dot(x[...], w1[...],
                               preferred_element_type=jnp.float32)


def _tc_b_body(s1p, g1, dinv, b1, g2_o):
    t = dinv[...] * (s1p[0] + s1p[1] - g1[...])
    z1 = jnp.maximum(t + b1[...], 0.0)
    g2_o[...] = dinv[...] * z1


def _tc_c_body(s2p, g2, dinv, w2, b2, g3_o):
    t = dinv[...] * (s2p[0] + s2p[1] - g2[...])
    z2 = jnp.maximum(jnp.dot(t, w2[...], preferred_element_type=jnp.float32)
                     + b2[...], 0.0)
    g3_o[...] = dinv[...] * z2


def _tc_d_body(s3p, g3, dinv, w3, b3, wfc, bfc, out_o):
    t = dinv[...] * (s3p[0] + s3p[1] - g3[...])
    z3 = jnp.maximum(jnp.dot(t, w3[...], preferred_element_type=jnp.float32)
                     + b3[...], 0.0)
    out_o[...] = jnp.dot(z3, wfc[...],
                         preferred_element_type=jnp.float32) + bfc[...]


def kernel(x, edge_index, W1, b1, W2, b2, W3, b3, Wfc, bfc):
    src3 = edge_index[0].reshape(NW, NCH, CH)
    dst3 = edge_index[1].reshape(NW, NCH, CH)

    ones8 = jnp.ones((N, 8), dtype=jnp.float32)
    degp = _seg8_const(ones8, src3, dst3)

    dinv, g1 = _tc_call(
        _tc_a_body,
        (jax.ShapeDtypeStruct((N, 1), jnp.float32),
         jax.ShapeDtypeStruct((N, 16), jnp.float32)))(degp, x, W1)

    s1p = _seg16(g1, src3, dst3)
    g2 = _tc_call(
        _tc_b_body,
        jax.ShapeDtypeStruct((N, 16), jnp.float32))(
            s1p, g1, dinv, b1.reshape(1, 16))

    s2p = _seg16(g2, src3, dst3)
    g3 = _tc_call(
        _tc_c_body,
        jax.ShapeDtypeStruct((N, 64), jnp.float32))(
            s2p, g2, dinv, W2, b2.reshape(1, 64))

    s3p = _seg64(g3, src3, dst3)
    out = _tc_call(
        _tc_d_body,
        jax.ShapeDtypeStruct((N, 1), jnp.float32))(
            s3p, g3, dinv, W3, b3.reshape(1, 128), Wfc, bfc.reshape(1, 1))
    return out
